# Initial kernel scaffold; baseline (speedup 1.0000x reference)
import jax
import jax.numpy as jnp
from jax.experimental import pallas as pl


def kernel(used_symbols, distribution, pos_encoding, symbol_embeddings):
    raise NotImplementedError("write your pallas kernel here")


# speedup vs baseline: 3.4864x; 3.4864x over previous
import functools

import jax
import jax.numpy as jnp
from jax import lax
from jax.experimental import pallas as pl
from jax.experimental.pallas import tpu as pltpu
from jax.experimental.pallas import tpu_sc as plsc

B = 1024
S = 1001
E = 64
NC = 2
NS = 16
NW = NC * NS
ROWS_PER_W = B // NW
NCHUNK = 8
CW = 126
CG = 128
S_PAD = 1016
TAIL = S - (NCHUNK - 1) * CW


def _emb_body(idx3, dist3, pos_hbm, tab, out, idx_v, dist_v, pos_v, buf0, buf1,
              sem0, sem1):
    wid = lax.axis_index("s") * NC + lax.axis_index("c")
    base = wid * ROWS_PER_W

    pltpu.sync_copy(pos_hbm, pos_v)

    ri = lax.iota(jnp.int32, 16)
    col63 = jnp.full((16,), E - 1, jnp.int32)
    bufs = [buf0, buf1]
    sems = [sem0, sem1]

    def row_body(r, carry):
        b = base + r
        pltpu.sync_copy(idx3.at[b], idx_v)
        pltpu.sync_copy(dist3.at[b], dist_v)

        copies = {}
        copies[0] = pltpu.async_copy(tab.at[idx_v.at[0]], buf0, sem0)
        for j in range(NCHUNK):
            if j + 1 < NCHUNK:
                copies[j + 1] = pltpu.async_copy(
                    tab.at[idx_v.at[j + 1]], bufs[(j + 1) % 2], sems[(j + 1) % 2])
            copies[j].wait()
            bufp = bufs[j % 2]

            def add_pos(i, c, _j=j, _bufp=bufp):
                for cc in range(E // 16):
                    sl = pl.ds(cc * 16, 16)
                    _bufp[i, sl] = _bufp[i, sl] + pos_v[_j * CW + i, sl]
                return c

            lax.fori_loop(0, CG, add_pos, 0)

            for t in range(CG // 16):
                plsc.addupdate_scatter(
                    bufp, [t * 16 + ri, col63], dist_v[j, pl.ds(t * 16, 16)])

            rows = CW if j + 1 < NCHUNK else TAIL
            pltpu.sync_copy(bufp.at[pl.ds(0, rows)],
                            out.at[b].at[pl.ds(j * CW, rows)])
        return carry

    lax.fori_loop(0, ROWS_PER_W, row_body, 0)


@functools.partial(jax.jit, static_argnames=())
def kernel(used_symbols, distribution, pos_encoding, symbol_embeddings):
    u = used_symbols[:, :S].astype(jnp.int32)
    u_pad = jnp.pad(u, ((0, 0), (0, S_PAD - S)))
    idx3 = jnp.stack([u_pad[:, j * CW:j * CW + CG]
                      for j in range(NCHUNK)], axis=1)
    d_pad = jnp.pad(distribution, ((0, 0), (0, S_PAD - S)))
    dist3 = jnp.stack([d_pad[:, j * CW:j * CW + CG]
                       for j in range(NCHUNK)], axis=1)
    pos_pad = jnp.pad(pos_encoding, ((0, S_PAD - S), (0, 0)))
    tab = jnp.pad(symbol_embeddings, ((0, 0), (0, 1)))

    mesh = plsc.VectorSubcoreMesh(core_axis_name="c", subcore_axis_name="s")
    run = pl.kernel(
        _emb_body,
        out_type=jax.ShapeDtypeStruct((B, S, E), jnp.float32),
        mesh=mesh,
        scratch_types=[
            pltpu.VMEM((NCHUNK, CG), jnp.int32),
            pltpu.VMEM((NCHUNK, CG), jnp.float32),
            pltpu.VMEM((S_PAD, E), jnp.float32),
            pltpu.VMEM((CG, E), jnp.float32),
            pltpu.VMEM((CG, E), jnp.float32),
            pltpu.SemaphoreType.DMA,
            pltpu.SemaphoreType.DMA,
        ],
        compiler_params=pltpu.CompilerParams(use_tc_tiling_on_sc=False,
                                             needs_layout_passes=False),
    )
    return run(idx3, dist3, pos_pad, tab)
